# Initial kernel scaffold; baseline (speedup 1.0000x reference)
import jax
import jax.numpy as jnp
from jax.experimental import pallas as pl


def kernel(x, adj_t, W1_0, b1_0, g_0, bt_0, W2_0, b2_0, W1_1, b1_1, g_1, bt_1, W2_1, b2_1, W1_2, b1_2, g_2, bt_2, W2_2, b2_2):
    raise NotImplementedError("write your pallas kernel here")


# speedup vs baseline: 3.9124x; 3.9124x over previous
import functools
import math

import jax
import jax.numpy as jnp
from jax import lax
from jax.experimental import pallas as pl
from jax.experimental.pallas import tpu as pltpu
from jax.experimental.pallas import tpu_sc as plsc

_N = 10000
_H = 128
_E = 320000
_L = 3
_BN_EPS = 1e-5

_NC = 2
_NS = 16
_NW = _NC * _NS
_CHUNK = 128
_KCH = 79
_EPW = _KCH * _CHUNK
_EPAD = _EPW * _NW
_NPAD = 10240
_RPT = _NPAD // _NS


def _sc_agg_body(h_hbm, src_hbm, dst_hbm, zeros_hbm, out_hbm,
                 src_v, dst_v, rows_v, acc, gsem):
    cid = lax.axis_index("c")
    sid = lax.axis_index("s")
    wid = cid * _NS + sid
    pltpu.sync_copy(zeros_hbm.at[pl.ds(sid * _RPT, _RPT)],
                    acc.at[pl.ds(sid * _RPT, _RPT)])
    pltpu.sync_copy(src_hbm.at[wid], src_v)
    pltpu.sync_copy(dst_hbm.at[wid], dst_v)
    plsc.subcore_barrier()

    def step(j, carry):
        pltpu.async_copy(h_hbm.at[src_v.at[j]], rows_v, gsem).wait()
        pltpu.sync_copy(rows_v, acc.at[dst_v.at[j]], add=True)
        return carry

    lax.fori_loop(0, _KCH, step, 0)
    plsc.subcore_barrier()
    pltpu.sync_copy(acc.at[pl.ds(sid * _RPT, _RPT)],
                    out_hbm.at[cid, pl.ds(sid * _RPT, _RPT)])


_sc_agg = functools.partial(
    pl.kernel,
    out_type=jax.ShapeDtypeStruct((_NC, _NPAD, _H), jnp.float32),
    mesh=plsc.VectorSubcoreMesh(core_axis_name="c", subcore_axis_name="s"),
    scratch_types=[
        pltpu.VMEM((_KCH, _CHUNK), jnp.int32),
        pltpu.VMEM((_KCH, _CHUNK), jnp.int32),
        pltpu.VMEM((_CHUNK, _H), jnp.float32),
        pltpu.VMEM_SHARED((_NPAD, _H), jnp.float32),
        pltpu.SemaphoreType.DMA,
    ],
)(_sc_agg_body)


_ROWBLK = 256
_BN_INV = 1.0 / math.sqrt(1.0 + _BN_EPS)


def _mlp_body(x_ref, a0_ref, a1_ref, w1_ref, b1_ref, g_ref, bt_ref,
              w2_ref, b2_ref, o_ref):
    h = x_ref[...] + a0_ref[...] + a1_ref[...]
    t = jnp.dot(h, w1_ref[...], preferred_element_type=jnp.float32)
    t = (t + b1_ref[...]) * (g_ref[...] * _BN_INV) + bt_ref[...]
    t = jnp.maximum(t, 0.0)
    u = jnp.dot(t, w2_ref[...], preferred_element_type=jnp.float32)
    o_ref[...] = jnp.maximum(u + b2_ref[...], 0.0)


def _mlp(x, a0, a1, w1, b1, g, bt, w2, b2):
    grid = (_N + _ROWBLK - 1) // _ROWBLK
    row_spec = pl.BlockSpec((_ROWBLK, _H), lambda i: (i, 0))
    full_spec = pl.BlockSpec((_H, _H), lambda i: (0, 0))
    vec_spec = pl.BlockSpec((1, _H), lambda i: (0, 0))
    return pl.pallas_call(
        _mlp_body,
        grid=(grid,),
        in_specs=[row_spec, row_spec, row_spec,
                  full_spec, vec_spec, vec_spec, vec_spec,
                  full_spec, vec_spec],
        out_specs=row_spec,
        out_shape=jax.ShapeDtypeStruct((_N, _H), jnp.float32),
    )(x, a0, a1, w1, b1.reshape(1, _H), g.reshape(1, _H),
      bt.reshape(1, _H), w2, b2.reshape(1, _H))


def kernel(x, adj_t,
           W1_0, b1_0, g_0, bt_0, W2_0, b2_0,
           W1_1, b1_1, g_1, bt_1, W2_1, b2_1,
           W1_2, b1_2, g_2, bt_2, W2_2, b2_2):
    src = adj_t[0]
    dst = adj_t[1]
    pad = _EPAD - _E
    srcp = jnp.concatenate(
        [src, jnp.zeros((pad,), jnp.int32)]).reshape(_NW, _KCH, _CHUNK)
    dstp = jnp.concatenate(
        [dst, jnp.full((pad,), _N, jnp.int32)]).reshape(_NW, _KCH, _CHUNK)
    zeros = jnp.zeros((_NPAD, _H), jnp.float32)

    params = [
        (W1_0, b1_0, g_0, bt_0, W2_0, b2_0),
        (W1_1, b1_1, g_1, bt_1, W2_1, b2_1),
        (W1_2, b1_2, g_2, bt_2, W2_2, b2_2),
    ]
    h = x
    for (w1, b1, g, bt, w2, b2) in params:
        aggs = _sc_agg(h, srcp, dstp, zeros)
        h = _mlp(h, aggs[0], aggs[1], w1, b1, g, bt, w2, b2)
    return h
